# baseline (speedup 1.0000x reference)
import functools

import jax
import jax.numpy as jnp
from jax import lax
from jax.experimental import pallas as pl
from jax.experimental.pallas import tpu as pltpu, tpu_sc as plsc

_H = 128


def _sc_extract_body(x_hbm, bg_hbm, sem):
    wid = lax.axis_index("s") * 2 + lax.axis_index("c")

    @pl.when(wid == 0)
    def _():
        pltpu.async_copy(x_hbm.at[3, :, 0:16], bg_hbm, sem).wait()


def _main_body(si_ref, bg_ref, wt_ref, b_ref, out_ref, k_ref, *, block_rows):
    i = pl.program_id(0)
    B = block_rows
    blk3 = bg_ref[...]

    @pl.when(i == 0)
    def _():
        k_ref[0] = 0

    k_ref[0] += jnp.sum((blk3[:, 2:3] != 0.0).astype(jnp.int32))

    v = si_ref[...].reshape(2 * B, _H)
    sir = jax.nn.sigmoid(
        jax.lax.dot_general(
            v, wt_ref[...], (((1,), (0,)), ((), ())),
            preferred_element_type=jnp.float32,
        )
        + b_ref[...]
    )
    s = sir[0:B]
    ii = sir[B:2 * B]
    beta = blk3[:, 0:1]
    gamma = blk3[:, 1:2]
    ds = -beta * (ii * s)
    dr = gamma * ii
    out_ref[0] = ds
    out_ref[1] = -ds - dr
    out_ref[2] = dr
    out_ref[3] = jnp.zeros_like(ds)


def _fixup_body(k_ref, x_ref, wt_ref, b_ref, oin_ref, out_ref,
                si_v, x3_v, dsdi_v, sem_si, sem_x3, sem_out, *, block_rows):
    del oin_ref
    i = pl.program_id(0)
    B = block_rows
    k = k_ref[0]

    @pl.when(k < (i + 1) * B)
    def _():
        cs = pltpu.make_async_copy(
            x_ref.at[0:2, pl.ds(i * B, B), :], si_v, sem_si)
        c3 = pltpu.make_async_copy(
            x_ref.at[3, pl.ds(i * B, B), :], x3_v, sem_x3)
        cs.start()
        c3.start()
        cs.wait()
        c3.wait()
        v = si_v[...].reshape(2 * B, _H)
        sir = jax.nn.sigmoid(
            jax.lax.dot_general(
                v, wt_ref[...], (((1,), (0,)), ((), ())),
                preferred_element_type=jnp.float32,
            )
            + b_ref[...]
        )
        s = sir[0:B]
        ii = sir[B:2 * B]
        row = i * B + jax.lax.broadcasted_iota(jnp.int32, (B, 1), 0)
        mask = (row < k).astype(jnp.float32)
        beta = x3_v[:, 0:1]
        gamma = x3_v[:, 1:2]
        ds = -beta * (ii * mask * s)
        dr = gamma * ii
        dsdi_v[0] = ds
        dsdi_v[1] = -ds - dr
        co = pltpu.make_async_copy(
            dsdi_v, out_ref.at[0:2, pl.ds(i * B, B), :], sem_out)
        co.start()
        co.wait()


def kernel(t, x, W, b):
    del t
    n = x.shape[1]
    block_rows = 5000
    nb = n // block_rows
    wt = W.T
    b2 = b.reshape(1, _H)
    bg16 = pl.kernel(
        _sc_extract_body,
        out_type=jax.ShapeDtypeStruct((n, 16), jnp.float32),
        mesh=plsc.VectorSubcoreMesh(core_axis_name="c", subcore_axis_name="s"),
        scratch_types=[pltpu.SemaphoreType.DMA],
        compiler_params=pltpu.CompilerParams(use_tc_tiling_on_sc=False),
    )(x)
    out_unmasked, karr = pl.pallas_call(
        functools.partial(_main_body, block_rows=block_rows),
        grid=(nb,),
        in_specs=[
            pl.BlockSpec((2, block_rows, _H), lambda i: (0, i, 0)),
            pl.BlockSpec((block_rows, 16), lambda i: (i, 0)),
            pl.BlockSpec((_H, _H), lambda i: (0, 0)),
            pl.BlockSpec((1, _H), lambda i: (0, 0)),
        ],
        out_specs=[
            pl.BlockSpec((4, block_rows, _H), lambda i: (0, i, 0)),
            pl.BlockSpec(memory_space=pltpu.SMEM),
        ],
        out_shape=[
            jax.ShapeDtypeStruct((4, n, _H), jnp.float32),
            jax.ShapeDtypeStruct((1,), jnp.int32),
        ],
    )(x, bg16, wt, b2)
    out = pl.pallas_call(
        functools.partial(_fixup_body, block_rows=block_rows),
        grid=(nb,),
        in_specs=[
            pl.BlockSpec(memory_space=pltpu.SMEM),
            pl.BlockSpec(memory_space=pltpu.MemorySpace.HBM),
            pl.BlockSpec(memory_space=pltpu.MemorySpace.VMEM),
            pl.BlockSpec(memory_space=pltpu.MemorySpace.VMEM),
            pl.BlockSpec(memory_space=pltpu.MemorySpace.HBM),
        ],
        out_specs=pl.BlockSpec(memory_space=pltpu.MemorySpace.HBM),
        out_shape=jax.ShapeDtypeStruct((4, n, _H), jnp.float32),
        input_output_aliases={4: 0},
        scratch_shapes=[
            pltpu.VMEM((2, block_rows, _H), jnp.float32),
            pltpu.VMEM((block_rows, _H), jnp.float32),
            pltpu.VMEM((2, block_rows, _H), jnp.float32),
            pltpu.SemaphoreType.DMA,
            pltpu.SemaphoreType.DMA,
            pltpu.SemaphoreType.DMA,
        ],
    )(karr, x, wt, b2, out_unmasked)
    return out


# speedup vs baseline: 2.6459x; 2.6459x over previous
import functools

import jax
import jax.numpy as jnp
from jax import lax
from jax.experimental import pallas as pl
from jax.experimental.pallas import tpu as pltpu, tpu_sc as plsc

_H = 128


_NPAD = 51200
_BPW = _NPAD // 32


def _sc_extract_body(table_hbm, idx_hbm, out_hbm, idx_v, rows_v, sem):
    wid = lax.axis_index("s") * 2 + lax.axis_index("c")
    base = wid * _BPW
    pltpu.sync_copy(idx_hbm.at[pl.ds(base, _BPW)], idx_v)
    pltpu.async_copy(table_hbm.at[idx_v], rows_v, sem).wait()
    pltpu.sync_copy(rows_v, out_hbm.at[pl.ds(base, _BPW)])


def _main_body(si_ref, bg_ref, wt_ref, b_ref, out_ref, k_ref, *, block_rows):
    i = pl.program_id(0)
    B = block_rows
    blk3 = bg_ref[...]

    @pl.when(i == 0)
    def _():
        k_ref[0] = 0

    k_ref[0] += jnp.sum((blk3[:, 2:3] != 0.0).astype(jnp.int32))

    v = si_ref[...].reshape(2 * B, _H)
    sir = jax.nn.sigmoid(
        jax.lax.dot_general(
            v, wt_ref[...], (((1,), (0,)), ((), ())),
            preferred_element_type=jnp.float32,
        )
        + b_ref[...]
    )
    s = sir[0:B]
    ii = sir[B:2 * B]
    beta = blk3[:, 0:1]
    gamma = blk3[:, 1:2]
    ds = -beta * (ii * s)
    dr = gamma * ii
    out_ref[0] = ds
    out_ref[1] = -ds - dr
    out_ref[2] = dr
    out_ref[3] = jnp.zeros_like(ds)


def _fixup_body(k_ref, x_ref, wt_ref, b_ref, oin_ref, out_ref,
                si_v, x3_v, dsdi_v, sem_si, sem_x3, sem_out, *, block_rows):
    del oin_ref
    i = pl.program_id(0)
    B = block_rows
    k = k_ref[0]

    @pl.when(k < (i + 1) * B)
    def _():
        cs = pltpu.make_async_copy(
            x_ref.at[0:2, pl.ds(i * B, B), :], si_v, sem_si)
        c3 = pltpu.make_async_copy(
            x_ref.at[3, pl.ds(i * B, B), :], x3_v, sem_x3)
        cs.start()
        c3.start()
        cs.wait()
        c3.wait()
        v = si_v[...].reshape(2 * B, _H)
        sir = jax.nn.sigmoid(
            jax.lax.dot_general(
                v, wt_ref[...], (((1,), (0,)), ((), ())),
                preferred_element_type=jnp.float32,
            )
            + b_ref[...]
        )
        s = sir[0:B]
        ii = sir[B:2 * B]
        row = i * B + jax.lax.broadcasted_iota(jnp.int32, (B, 1), 0)
        mask = (row < k).astype(jnp.float32)
        beta = x3_v[:, 0:1]
        gamma = x3_v[:, 1:2]
        ds = -beta * (ii * mask * s)
        dr = gamma * ii
        dsdi_v[0] = ds
        dsdi_v[1] = -ds - dr
        co = pltpu.make_async_copy(
            dsdi_v, out_ref.at[0:2, pl.ds(i * B, B), :], sem_out)
        co.start()
        co.wait()


def kernel(t, x, W, b):
    del t
    n = x.shape[1]
    block_rows = 5000
    nb = n // block_rows
    wt = W.T
    b2 = b.reshape(1, _H)
    table = x.reshape(4 * n * _H // 16, 16)
    idx = jnp.minimum((3 * n + jnp.arange(_NPAD, dtype=jnp.int32)) * 8,
                      table.shape[0] - 1)
    bg16 = pl.kernel(
        _sc_extract_body,
        out_type=jax.ShapeDtypeStruct((_NPAD, 16), jnp.float32),
        mesh=plsc.VectorSubcoreMesh(core_axis_name="c", subcore_axis_name="s"),
        scratch_types=[
            pltpu.VMEM((_BPW,), jnp.int32),
            pltpu.VMEM((_BPW, 16), jnp.float32),
            pltpu.SemaphoreType.DMA,
        ],
        compiler_params=pltpu.CompilerParams(use_tc_tiling_on_sc=False),
    )(table, idx)
    out_unmasked, karr = pl.pallas_call(
        functools.partial(_main_body, block_rows=block_rows),
        grid=(nb,),
        in_specs=[
            pl.BlockSpec((2, block_rows, _H), lambda i: (0, i, 0)),
            pl.BlockSpec((block_rows, 16), lambda i: (i, 0)),
            pl.BlockSpec((_H, _H), lambda i: (0, 0)),
            pl.BlockSpec((1, _H), lambda i: (0, 0)),
        ],
        out_specs=[
            pl.BlockSpec((4, block_rows, _H), lambda i: (0, i, 0)),
            pl.BlockSpec(memory_space=pltpu.SMEM),
        ],
        out_shape=[
            jax.ShapeDtypeStruct((4, n, _H), jnp.float32),
            jax.ShapeDtypeStruct((1,), jnp.int32),
        ],
    )(x, bg16, wt, b2)
    out = pl.pallas_call(
        functools.partial(_fixup_body, block_rows=block_rows),
        grid=(nb,),
        in_specs=[
            pl.BlockSpec(memory_space=pltpu.SMEM),
            pl.BlockSpec(memory_space=pltpu.MemorySpace.HBM),
            pl.BlockSpec(memory_space=pltpu.MemorySpace.VMEM),
            pl.BlockSpec(memory_space=pltpu.MemorySpace.VMEM),
            pl.BlockSpec(memory_space=pltpu.MemorySpace.HBM),
        ],
        out_specs=pl.BlockSpec(memory_space=pltpu.MemorySpace.HBM),
        out_shape=jax.ShapeDtypeStruct((4, n, _H), jnp.float32),
        input_output_aliases={4: 0},
        scratch_shapes=[
            pltpu.VMEM((2, block_rows, _H), jnp.float32),
            pltpu.VMEM((block_rows, _H), jnp.float32),
            pltpu.VMEM((2, block_rows, _H), jnp.float32),
            pltpu.SemaphoreType.DMA,
            pltpu.SemaphoreType.DMA,
            pltpu.SemaphoreType.DMA,
        ],
    )(karr, x, wt, b2, out_unmasked)
    return out


# speedup vs baseline: 4.4613x; 1.6861x over previous
import functools

import jax
import jax.numpy as jnp
from jax.experimental import pallas as pl
from jax.experimental.pallas import tpu as pltpu

_H = 128


def _main_body(si_ref, x3_ref, wt_ref, b_ref, out_ref, k_ref, *, block_rows):
    i = pl.program_id(0)
    B = block_rows
    blk3 = x3_ref[0]

    @pl.when(i == 0)
    def _():
        k_ref[0] = 0

    k_ref[0] += jnp.sum((blk3[:, 2:3] != 0.0).astype(jnp.int32))

    v = si_ref[...].reshape(2 * B, _H)
    sir = jax.nn.sigmoid(
        jax.lax.dot_general(
            v, wt_ref[...], (((1,), (0,)), ((), ())),
            preferred_element_type=jnp.float32,
        )
        + b_ref[...]
    )
    s = sir[0:B]
    ii = sir[B:2 * B]
    beta = blk3[:, 0:1]
    gamma = blk3[:, 1:2]
    ds = -beta * (ii * s)
    dr = gamma * ii
    out_ref[0] = ds
    out_ref[1] = -ds - dr
    out_ref[2] = dr
    out_ref[3] = jnp.zeros_like(ds)


def _fixup_body(k_ref, x_ref, wt_ref, b_ref, oin_ref, out_ref,
                si_v, x3_v, dsdi_v, sem_si, sem_x3, sem_out, *, block_rows):
    del oin_ref
    i = pl.program_id(0)
    B = block_rows
    k = k_ref[0]

    @pl.when(k < (i + 1) * B)
    def _():
        cs = pltpu.make_async_copy(
            x_ref.at[0:2, pl.ds(i * B, B), :], si_v, sem_si)
        c3 = pltpu.make_async_copy(
            x_ref.at[3, pl.ds(i * B, B), :], x3_v, sem_x3)
        cs.start()
        c3.start()
        cs.wait()
        c3.wait()
        v = si_v[...].reshape(2 * B, _H)
        sir = jax.nn.sigmoid(
            jax.lax.dot_general(
                v, wt_ref[...], (((1,), (0,)), ((), ())),
                preferred_element_type=jnp.float32,
            )
            + b_ref[...]
        )
        s = sir[0:B]
        ii = sir[B:2 * B]
        row = i * B + jax.lax.broadcasted_iota(jnp.int32, (B, 1), 0)
        mask = (row < k).astype(jnp.float32)
        beta = x3_v[:, 0:1]
        gamma = x3_v[:, 1:2]
        ds = -beta * (ii * mask * s)
        dr = gamma * ii
        dsdi_v[0] = ds
        dsdi_v[1] = -ds - dr
        co = pltpu.make_async_copy(
            dsdi_v, out_ref.at[0:2, pl.ds(i * B, B), :], sem_out)
        co.start()
        co.wait()


def kernel(t, x, W, b):
    del t
    n = x.shape[1]
    block_rows = 2000
    nb = n // block_rows
    wt = W.T
    b2 = b.reshape(1, _H)
    out_unmasked, karr = pl.pallas_call(
        functools.partial(_main_body, block_rows=block_rows),
        grid=(nb,),
        in_specs=[
            pl.BlockSpec((2, block_rows, _H), lambda i: (0, i, 0)),
            pl.BlockSpec((1, block_rows, _H), lambda i: (3, i, 0)),
            pl.BlockSpec((_H, _H), lambda i: (0, 0)),
            pl.BlockSpec((1, _H), lambda i: (0, 0)),
        ],
        out_specs=[
            pl.BlockSpec((4, block_rows, _H), lambda i: (0, i, 0)),
            pl.BlockSpec(memory_space=pltpu.SMEM),
        ],
        out_shape=[
            jax.ShapeDtypeStruct((4, n, _H), jnp.float32),
            jax.ShapeDtypeStruct((1,), jnp.int32),
        ],
    )(x, x, wt, b2)
    out = pl.pallas_call(
        functools.partial(_fixup_body, block_rows=block_rows),
        grid=(nb,),
        in_specs=[
            pl.BlockSpec(memory_space=pltpu.SMEM),
            pl.BlockSpec(memory_space=pltpu.MemorySpace.HBM),
            pl.BlockSpec(memory_space=pltpu.MemorySpace.VMEM),
            pl.BlockSpec(memory_space=pltpu.MemorySpace.VMEM),
            pl.BlockSpec(memory_space=pltpu.MemorySpace.HBM),
        ],
        out_specs=pl.BlockSpec(memory_space=pltpu.MemorySpace.HBM),
        out_shape=jax.ShapeDtypeStruct((4, n, _H), jnp.float32),
        input_output_aliases={4: 0},
        scratch_shapes=[
            pltpu.VMEM((2, block_rows, _H), jnp.float32),
            pltpu.VMEM((block_rows, _H), jnp.float32),
            pltpu.VMEM((2, block_rows, _H), jnp.float32),
            pltpu.SemaphoreType.DMA,
            pltpu.SemaphoreType.DMA,
            pltpu.SemaphoreType.DMA,
        ],
    )(karr, x, wt, b2, out_unmasked)
    return out


# speedup vs baseline: 4.6642x; 1.0455x over previous
import functools

import jax
import jax.numpy as jnp
from jax.experimental import pallas as pl
from jax.experimental.pallas import tpu as pltpu

_H = 128


def _main_body(si_ref, x3_ref, wt_ref, b_ref, out_ref, k_ref, *, block_rows):
    i = pl.program_id(0)
    B = block_rows
    blk3 = x3_ref[0]

    @pl.when(i == 0)
    def _():
        k_ref[0] = 0

    k_ref[0] += jnp.sum((blk3[:, 2:3] != 0.0).astype(jnp.int32))

    v = si_ref[...].reshape(2 * B, _H)
    sir = jax.nn.sigmoid(
        jax.lax.dot_general(
            v, wt_ref[...], (((1,), (0,)), ((), ())),
            preferred_element_type=jnp.float32,
        )
        + b_ref[...]
    )
    s = sir[0:B]
    ii = sir[B:2 * B]
    beta = blk3[:, 0:1]
    gamma = blk3[:, 1:2]
    ds = -beta * (ii * s)
    dr = gamma * ii
    out_ref[0] = ds
    out_ref[1] = -ds - dr
    out_ref[2] = dr
    out_ref[3] = jnp.zeros_like(ds)


def _fixup_body(k_ref, x_ref, wt_ref, b_ref, oin_ref, out_ref,
                si_v, x3_v, dsdi_v, sem_si, sem_x3, sem_out, *, block_rows):
    del oin_ref
    i = pl.program_id(0)
    B = block_rows
    k = k_ref[0]

    @pl.when(k < (i + 1) * B)
    def _():
        cs = pltpu.make_async_copy(
            x_ref.at[0:2, pl.ds(i * B, B), :], si_v, sem_si)
        c3 = pltpu.make_async_copy(
            x_ref.at[3, pl.ds(i * B, B), :], x3_v, sem_x3)
        cs.start()
        c3.start()
        cs.wait()
        c3.wait()
        v = si_v[...].reshape(2 * B, _H)
        sir = jax.nn.sigmoid(
            jax.lax.dot_general(
                v, wt_ref[...], (((1,), (0,)), ((), ())),
                preferred_element_type=jnp.float32,
            )
            + b_ref[...]
        )
        s = sir[0:B]
        ii = sir[B:2 * B]
        row = i * B + jax.lax.broadcasted_iota(jnp.int32, (B, 1), 0)
        mask = (row < k).astype(jnp.float32)
        beta = x3_v[:, 0:1]
        gamma = x3_v[:, 1:2]
        ds = -beta * (ii * mask * s)
        dr = gamma * ii
        dsdi_v[0] = ds
        dsdi_v[1] = -ds - dr
        co = pltpu.make_async_copy(
            dsdi_v, out_ref.at[0:2, pl.ds(i * B, B), :], sem_out)
        co.start()
        co.wait()


def kernel(t, x, W, b):
    del t
    n = x.shape[1]
    block_rows = 5000
    nb = n // block_rows
    wt = W.T
    b2 = b.reshape(1, _H)
    out_unmasked, karr = pl.pallas_call(
        functools.partial(_main_body, block_rows=block_rows),
        grid=(nb,),
        in_specs=[
            pl.BlockSpec((2, block_rows, _H), lambda i: (0, i, 0)),
            pl.BlockSpec((1, block_rows, _H), lambda i: (3, i, 0)),
            pl.BlockSpec((_H, _H), lambda i: (0, 0)),
            pl.BlockSpec((1, _H), lambda i: (0, 0)),
        ],
        out_specs=[
            pl.BlockSpec((4, block_rows, _H), lambda i: (0, i, 0)),
            pl.BlockSpec(memory_space=pltpu.SMEM),
        ],
        out_shape=[
            jax.ShapeDtypeStruct((4, n, _H), jnp.float32),
            jax.ShapeDtypeStruct((1,), jnp.int32),
        ],
    )(x, x, wt, b2)
    out = pl.pallas_call(
        functools.partial(_fixup_body, block_rows=block_rows),
        grid=(nb,),
        in_specs=[
            pl.BlockSpec(memory_space=pltpu.SMEM),
            pl.BlockSpec(memory_space=pltpu.MemorySpace.HBM),
            pl.BlockSpec(memory_space=pltpu.MemorySpace.VMEM),
            pl.BlockSpec(memory_space=pltpu.MemorySpace.VMEM),
            pl.BlockSpec(memory_space=pltpu.MemorySpace.HBM),
        ],
        out_specs=pl.BlockSpec(memory_space=pltpu.MemorySpace.HBM),
        out_shape=jax.ShapeDtypeStruct((4, n, _H), jnp.float32),
        input_output_aliases={4: 0},
        scratch_shapes=[
            pltpu.VMEM((2, block_rows, _H), jnp.float32),
            pltpu.VMEM((block_rows, _H), jnp.float32),
            pltpu.VMEM((2, block_rows, _H), jnp.float32),
            pltpu.SemaphoreType.DMA,
            pltpu.SemaphoreType.DMA,
            pltpu.SemaphoreType.DMA,
        ],
    )(karr, x, wt, b2, out_unmasked)
    return out
